# baseline (speedup 1.0000x reference)
import jax
import jax.numpy as jnp
from jax.experimental import pallas as pl
from jax.experimental.pallas import tpu as pltpu

N = 8192
TILE = 512
NGRID = N // TILE
F32 = jnp.float32


def _dot(a, b):
    return jax.lax.dot(a, b, preferred_element_type=F32)


def _dot_t(w, x):
    return jax.lax.dot_general(w, x, (((0,), (1,)), ((), ())),
                               preferred_element_type=F32)


def _stream_kernel(x1_ref, x2_ref,
                   pvw_ref, pvb_ref, fc1w_ref, fc1b_ref,
                   a1aw_ref, a1ab_ref, a1bw_ref, a1bb_ref, a1cw_ref, a1cb_ref,
                   fc2w_ref, fc2b_ref,
                   a2aw_ref, a2ab_ref, a2bw_ref, a2bb_ref, a2cw_ref, a2cb_ref,
                   a1_out, a2_out, m1_out, m2_out,
                   m1s, l1s, acc1, m2s, l2s, acc2):
    i = pl.program_id(0)

    @pl.when(i == 0)
    def _init():
        m1s[...] = jnp.full((2, 1), -jnp.inf, F32)
        l1s[...] = jnp.zeros((2, 1), F32)
        acc1[...] = jnp.zeros((2, 512), F32)
        m2s[...] = jnp.full((2, 1), -jnp.inf, F32)
        l2s[...] = jnp.zeros((2, 1), F32)
        acc2[...] = jnp.zeros((2, 512), F32)

    bf16 = jnp.bfloat16

    h1 = _dot(x1_ref[...].astype(bf16), pvw_ref[...]) + pvb_ref[...]
    h1 = jnp.maximum(_dot(h1.astype(bf16), fc1w_ref[...]) + fc1b_ref[...], 0.0)
    h1b = h1.astype(bf16)
    a = jnp.tanh(_dot(h1b, a1aw_ref[...]) + a1ab_ref[...])
    b = jax.nn.sigmoid(_dot(h1b, a1bw_ref[...]) + a1bb_ref[...])
    a1t = _dot_t(a1cw_ref[...], a * b) + a1cb_ref[...]
    a1_out[...] = a1t

    h2 = jnp.maximum(_dot(x2_ref[...].astype(bf16), fc2w_ref[...])
                     + fc2b_ref[...], 0.0)
    h2b = h2.astype(bf16)
    a = jnp.tanh(_dot(h2b, a2aw_ref[...]) + a2ab_ref[...])
    b = jax.nn.sigmoid(_dot(h2b, a2bw_ref[...]) + a2bb_ref[...])
    a2t = _dot_t(a2cw_ref[...], a * b) + a2cb_ref[...]
    a2_out[...] = a2t

    t1 = jnp.maximum(jnp.max(a1t, axis=1, keepdims=True), m1s[...])
    c1 = jnp.exp(m1s[...] - t1)
    p1 = jnp.exp(a1t - t1)
    l1s[...] = l1s[...] * c1 + jnp.sum(p1, axis=1, keepdims=True)
    acc1[...] = acc1[...] * c1 + _dot(p1.astype(bf16), h1b)
    m1s[...] = t1

    t2 = jnp.maximum(jnp.max(a2t, axis=1, keepdims=True), m2s[...])
    c2 = jnp.exp(m2s[...] - t2)
    p2 = jnp.exp(a2t - t2)
    l2s[...] = l2s[...] * c2 + jnp.sum(p2, axis=1, keepdims=True)
    acc2[...] = acc2[...] * c2 + _dot(p2.astype(bf16), h2b)
    m2s[...] = t2

    @pl.when(i == NGRID - 1)
    def _fin():
        m1_out[...] = acc1[...] / l1s[...]
        m2_out[...] = acc2[...] / l2s[...]


def _tail_kernel(m1_ref, m2_ref,
                 e1w1_ref, e1b1_ref, e1w2_ref, e1b2_ref,
                 e3w1_ref, e3b1_ref, e3w2_ref, e3b2_ref,
                 e2w1_ref, e2b1_ref, e2w2_ref, e2b2_ref, e2pw_ref, e2pb_ref,
                 gw1_ref, gb1_ref, gw2_ref, gb2_ref,
                 fw_ref, fb_ref, cw_ref, cb_ref,
                 logits_out, prob_out, yhat_out):
    m1 = m1_ref[...]
    m2 = m2_ref[...]
    relu = lambda v: jnp.maximum(v, 0.0)
    e1 = relu(_dot(relu(_dot(m1, e1w1_ref[...]) + e1b1_ref[...]),
                   e1w2_ref[...]) + e1b2_ref[...])
    e3 = relu(_dot(relu(_dot(m2, e3w1_ref[...]) + e3b1_ref[...]),
                   e3w2_ref[...]) + e3b2_ref[...])
    t = relu(_dot(m1, e2w1_ref[0:512, :]) + _dot(m2, e2w1_ref[512:1024, :])
             + e2b1_ref[...])
    e2 = relu(_dot(t, e2w2_ref[...]) + e2b2_ref[...])
    e2 = _dot(e2, e2pw_ref[...]) + e2pb_ref[...]
    g = (_dot(e1, gw1_ref[0:512, :]) + _dot(e2, gw1_ref[512:1024, :])
         + _dot(e3, gw1_ref[1024:1536, :]) + gb1_ref[...])
    g = _dot(relu(g), gw2_ref[...]) + gb2_ref[...]
    g = g - jnp.max(g, axis=1, keepdims=True)
    g = jnp.exp(g)
    g = g / jnp.sum(g, axis=1, keepdims=True)
    fused = (g[:, 0:1] * e1 + g[:, 1:2] * e2 + g[:, 2:3] * e3
             + _dot(m1, fw_ref[0:512, :]) + _dot(m2, fw_ref[512:1024, :])
             + fb_ref[...])
    logits = _dot(fused, cw_ref[...]) + cb_ref[...]
    logits_out[...] = logits
    z = logits - jnp.max(logits, axis=1, keepdims=True)
    z = jnp.exp(z)
    prob_out[...] = z / jnp.sum(z, axis=1, keepdims=True)
    yhat_out[...] = (logits[:, 1:2] > logits[:, 0:1]).astype(jnp.int32)


def _full(shape):
    return pl.BlockSpec(shape, lambda i: (0, 0))


def kernel(x1, x2, params):
    p = params
    row = lambda v: v.reshape(1, -1)
    col = lambda v: v.reshape(-1, 1)

    bf = lambda v: v.astype(jnp.bfloat16)
    stream_in = [
        x1, x2,
        bf(p['proj_virchow_w']), row(p['proj_virchow_b']),
        bf(p['fc1_w']), row(p['fc1_b']),
        bf(p['attn1_a_w']), row(p['attn1_a_b']),
        bf(p['attn1_b_w']), row(p['attn1_b_b']),
        p['attn1_c_w'], col(p['attn1_c_b']),
        bf(p['fc2_w']), row(p['fc2_b']),
        bf(p['attn2_a_w']), row(p['attn2_a_b']),
        bf(p['attn2_b_w']), row(p['attn2_b_b']),
        p['attn2_c_w'], col(p['attn2_c_b']),
    ]
    stream_specs = [
        pl.BlockSpec((TILE, 2560), lambda i: (i, 0)),
        pl.BlockSpec((TILE, 1024), lambda i: (i, 0)),
    ] + [_full(a.shape) for a in stream_in[2:]]

    a1_raw, a2_raw, m1, m2 = pl.pallas_call(
        _stream_kernel,
        grid=(NGRID,),
        in_specs=stream_specs,
        out_specs=[
            pl.BlockSpec((2, TILE), lambda i: (0, i)),
            pl.BlockSpec((2, TILE), lambda i: (0, i)),
            _full((2, 512)),
            _full((2, 512)),
        ],
        out_shape=[
            jax.ShapeDtypeStruct((2, N), F32),
            jax.ShapeDtypeStruct((2, N), F32),
            jax.ShapeDtypeStruct((2, 512), F32),
            jax.ShapeDtypeStruct((2, 512), F32),
        ],
        scratch_shapes=[
            pltpu.VMEM((2, 1), F32), pltpu.VMEM((2, 1), F32),
            pltpu.VMEM((2, 512), F32),
            pltpu.VMEM((2, 1), F32), pltpu.VMEM((2, 1), F32),
            pltpu.VMEM((2, 512), F32),
        ],
    )(*stream_in)

    tail_in = [
        m1, m2,
        p['e1_w1'], row(p['e1_b1']), p['e1_w2'], row(p['e1_b2']),
        p['e3_w1'], row(p['e3_b1']), p['e3_w2'], row(p['e3_b2']),
        p['e2_w1'], row(p['e2_b1']), p['e2_w2'], row(p['e2_b2']),
        p['e2_proj_w'], row(p['e2_proj_b']),
        p['gate_w1'], row(p['gate_b1']), p['gate_w2'], row(p['gate_b2']),
        p['fusion_w'], row(p['fusion_b']),
        p['cls_w'], row(p['cls_b']),
    ]
    logits, y_prob, yhat = pl.pallas_call(
        _tail_kernel,
        out_shape=[
            jax.ShapeDtypeStruct((2, 2), F32),
            jax.ShapeDtypeStruct((2, 2), F32),
            jax.ShapeDtypeStruct((2, 1), jnp.int32),
        ],
    )(*tail_in)

    return (logits, y_prob, yhat.reshape(2), a1_raw, a2_raw)


# speedup vs baseline: 1.0683x; 1.0683x over previous
import jax
import jax.numpy as jnp
from jax.experimental import pallas as pl
from jax.experimental.pallas import tpu as pltpu

N = 8192
TILE = 512
NGRID = N // TILE
F32 = jnp.float32


def _dot(a, b):
    return jax.lax.dot(a, b, preferred_element_type=F32)


def _dot_t(w, x):
    return jax.lax.dot_general(w, x, (((0,), (1,)), ((), ())),
                               preferred_element_type=F32)


def _stream_kernel(x1_ref, x2_ref,
                   pvw_ref, pvb_ref, fc1w_ref, fc1b_ref,
                   a1aw_ref, a1ab_ref, a1bw_ref, a1bb_ref, a1cw_ref, a1cb_ref,
                   fc2w_ref, fc2b_ref,
                   a2aw_ref, a2ab_ref, a2bw_ref, a2bb_ref, a2cw_ref, a2cb_ref,
                   a1_out, a2_out, m1_out, m2_out,
                   m1s, l1s, acc1, m2s, l2s, acc2):
    i = pl.program_id(0)

    @pl.when(i == 0)
    def _init():
        m1s[...] = jnp.full((2, 1), -jnp.inf, F32)
        l1s[...] = jnp.zeros((2, 1), F32)
        acc1[...] = jnp.zeros((2, 512), F32)
        m2s[...] = jnp.full((2, 1), -jnp.inf, F32)
        l2s[...] = jnp.zeros((2, 1), F32)
        acc2[...] = jnp.zeros((2, 512), F32)

    h1 = _dot(x1_ref[...], pvw_ref[...]) + pvb_ref[...]
    h1 = jnp.maximum(_dot(h1, fc1w_ref[...]) + fc1b_ref[...], 0.0)
    a = jnp.tanh(_dot(h1, a1aw_ref[...]) + a1ab_ref[...])
    b = jax.nn.sigmoid(_dot(h1, a1bw_ref[...]) + a1bb_ref[...])
    a1t = _dot_t(a1cw_ref[...], a * b) + a1cb_ref[...]
    a1_out[...] = a1t

    h2 = jnp.maximum(_dot(x2_ref[...], fc2w_ref[...]) + fc2b_ref[...], 0.0)
    a = jnp.tanh(_dot(h2, a2aw_ref[...]) + a2ab_ref[...])
    b = jax.nn.sigmoid(_dot(h2, a2bw_ref[...]) + a2bb_ref[...])
    a2t = _dot_t(a2cw_ref[...], a * b) + a2cb_ref[...]
    a2_out[...] = a2t

    t1 = jnp.maximum(jnp.max(a1t, axis=1, keepdims=True), m1s[...])
    c1 = jnp.exp(m1s[...] - t1)
    p1 = jnp.exp(a1t - t1)
    l1s[...] = l1s[...] * c1 + jnp.sum(p1, axis=1, keepdims=True)
    acc1[...] = acc1[...] * c1 + _dot(p1, h1)
    m1s[...] = t1

    t2 = jnp.maximum(jnp.max(a2t, axis=1, keepdims=True), m2s[...])
    c2 = jnp.exp(m2s[...] - t2)
    p2 = jnp.exp(a2t - t2)
    l2s[...] = l2s[...] * c2 + jnp.sum(p2, axis=1, keepdims=True)
    acc2[...] = acc2[...] * c2 + _dot(p2, h2)
    m2s[...] = t2

    @pl.when(i == NGRID - 1)
    def _fin():
        m1_out[...] = acc1[...] / l1s[...]
        m2_out[...] = acc2[...] / l2s[...]


def _tail_kernel(m1_ref, m2_ref,
                 e1w1_ref, e1b1_ref, e1w2_ref, e1b2_ref,
                 e3w1_ref, e3b1_ref, e3w2_ref, e3b2_ref,
                 e2w1_ref, e2b1_ref, e2w2_ref, e2b2_ref, e2pw_ref, e2pb_ref,
                 gw1_ref, gb1_ref, gw2_ref, gb2_ref,
                 fw_ref, fb_ref, cw_ref, cb_ref,
                 logits_out, prob_out, yhat_out):
    m1 = m1_ref[...]
    m2 = m2_ref[...]
    relu = lambda v: jnp.maximum(v, 0.0)
    e1 = relu(_dot(relu(_dot(m1, e1w1_ref[...]) + e1b1_ref[...]),
                   e1w2_ref[...]) + e1b2_ref[...])
    e3 = relu(_dot(relu(_dot(m2, e3w1_ref[...]) + e3b1_ref[...]),
                   e3w2_ref[...]) + e3b2_ref[...])
    t = relu(_dot(m1, e2w1_ref[0:512, :]) + _dot(m2, e2w1_ref[512:1024, :])
             + e2b1_ref[...])
    e2 = relu(_dot(t, e2w2_ref[...]) + e2b2_ref[...])
    e2 = _dot(e2, e2pw_ref[...]) + e2pb_ref[...]
    g = (_dot(e1, gw1_ref[0:512, :]) + _dot(e2, gw1_ref[512:1024, :])
         + _dot(e3, gw1_ref[1024:1536, :]) + gb1_ref[...])
    g = _dot(relu(g), gw2_ref[...]) + gb2_ref[...]
    g = g - jnp.max(g, axis=1, keepdims=True)
    g = jnp.exp(g)
    g = g / jnp.sum(g, axis=1, keepdims=True)
    fused = (g[:, 0:1] * e1 + g[:, 1:2] * e2 + g[:, 2:3] * e3
             + _dot(m1, fw_ref[0:512, :]) + _dot(m2, fw_ref[512:1024, :])
             + fb_ref[...])
    logits = _dot(fused, cw_ref[...]) + cb_ref[...]
    logits_out[...] = logits
    z = logits - jnp.max(logits, axis=1, keepdims=True)
    z = jnp.exp(z)
    prob_out[...] = z / jnp.sum(z, axis=1, keepdims=True)
    yhat_out[...] = (logits[:, 1:2] > logits[:, 0:1]).astype(jnp.int32)


def _full(shape):
    return pl.BlockSpec(shape, lambda i: (0, 0))


def kernel(x1, x2, params):
    p = params
    row = lambda v: v.reshape(1, -1)
    col = lambda v: v.reshape(-1, 1)

    stream_in = [
        x1, x2,
        p['proj_virchow_w'], row(p['proj_virchow_b']),
        p['fc1_w'], row(p['fc1_b']),
        p['attn1_a_w'], row(p['attn1_a_b']),
        p['attn1_b_w'], row(p['attn1_b_b']),
        p['attn1_c_w'], col(p['attn1_c_b']),
        p['fc2_w'], row(p['fc2_b']),
        p['attn2_a_w'], row(p['attn2_a_b']),
        p['attn2_b_w'], row(p['attn2_b_b']),
        p['attn2_c_w'], col(p['attn2_c_b']),
    ]
    stream_specs = [
        pl.BlockSpec((TILE, 2560), lambda i: (i, 0)),
        pl.BlockSpec((TILE, 1024), lambda i: (i, 0)),
    ] + [_full(a.shape) for a in stream_in[2:]]

    a1_raw, a2_raw, m1, m2 = pl.pallas_call(
        _stream_kernel,
        grid=(NGRID,),
        in_specs=stream_specs,
        out_specs=[
            pl.BlockSpec((2, TILE), lambda i: (0, i)),
            pl.BlockSpec((2, TILE), lambda i: (0, i)),
            _full((2, 512)),
            _full((2, 512)),
        ],
        out_shape=[
            jax.ShapeDtypeStruct((2, N), F32),
            jax.ShapeDtypeStruct((2, N), F32),
            jax.ShapeDtypeStruct((2, 512), F32),
            jax.ShapeDtypeStruct((2, 512), F32),
        ],
        scratch_shapes=[
            pltpu.VMEM((2, 1), F32), pltpu.VMEM((2, 1), F32),
            pltpu.VMEM((2, 512), F32),
            pltpu.VMEM((2, 1), F32), pltpu.VMEM((2, 1), F32),
            pltpu.VMEM((2, 512), F32),
        ],
    )(*stream_in)

    tail_in = [
        m1, m2,
        p['e1_w1'], row(p['e1_b1']), p['e1_w2'], row(p['e1_b2']),
        p['e3_w1'], row(p['e3_b1']), p['e3_w2'], row(p['e3_b2']),
        p['e2_w1'], row(p['e2_b1']), p['e2_w2'], row(p['e2_b2']),
        p['e2_proj_w'], row(p['e2_proj_b']),
        p['gate_w1'], row(p['gate_b1']), p['gate_w2'], row(p['gate_b2']),
        p['fusion_w'], row(p['fusion_b']),
        p['cls_w'], row(p['cls_b']),
    ]
    logits, y_prob, yhat = pl.pallas_call(
        _tail_kernel,
        out_shape=[
            jax.ShapeDtypeStruct((2, 2), F32),
            jax.ShapeDtypeStruct((2, 2), F32),
            jax.ShapeDtypeStruct((2, 1), jnp.int32),
        ],
    )(*tail_in)

    return (logits, y_prob, yhat.reshape(2), a1_raw, a2_raw)


# speedup vs baseline: 1.1027x; 1.0322x over previous
import jax
import jax.numpy as jnp
from jax.experimental import pallas as pl
from jax.experimental.pallas import tpu as pltpu

N = 8192
TILE = 1024
NGRID = N // TILE
F32 = jnp.float32


def _dot(a, b):
    return jax.lax.dot(a, b, preferred_element_type=F32)


def _dot_t(w, x):
    return jax.lax.dot_general(w, x, (((0,), (1,)), ((), ())),
                               preferred_element_type=F32)


def _stream_kernel(x1_ref, x2_ref,
                   pvw_ref, pvb_ref, fc1w_ref, fc1b_ref,
                   a1aw_ref, a1ab_ref, a1bw_ref, a1bb_ref, a1cw_ref, a1cb_ref,
                   fc2w_ref, fc2b_ref,
                   a2aw_ref, a2ab_ref, a2bw_ref, a2bb_ref, a2cw_ref, a2cb_ref,
                   a1_out, a2_out, m1_out, m2_out,
                   m1s, l1s, acc1, m2s, l2s, acc2):
    i = pl.program_id(0)

    @pl.when(i == 0)
    def _init():
        m1s[...] = jnp.full((2, 1), -jnp.inf, F32)
        l1s[...] = jnp.zeros((2, 1), F32)
        acc1[...] = jnp.zeros((2, 512), F32)
        m2s[...] = jnp.full((2, 1), -jnp.inf, F32)
        l2s[...] = jnp.zeros((2, 1), F32)
        acc2[...] = jnp.zeros((2, 512), F32)

    h1 = _dot(x1_ref[...], pvw_ref[...]) + pvb_ref[...]
    h1 = jnp.maximum(_dot(h1, fc1w_ref[...]) + fc1b_ref[...], 0.0)
    a = jnp.tanh(_dot(h1, a1aw_ref[...]) + a1ab_ref[...])
    b = jax.nn.sigmoid(_dot(h1, a1bw_ref[...]) + a1bb_ref[...])
    a1t = _dot_t(a1cw_ref[...], a * b) + a1cb_ref[...]
    a1_out[...] = a1t

    h2 = jnp.maximum(_dot(x2_ref[...], fc2w_ref[...]) + fc2b_ref[...], 0.0)
    a = jnp.tanh(_dot(h2, a2aw_ref[...]) + a2ab_ref[...])
    b = jax.nn.sigmoid(_dot(h2, a2bw_ref[...]) + a2bb_ref[...])
    a2t = _dot_t(a2cw_ref[...], a * b) + a2cb_ref[...]
    a2_out[...] = a2t

    t1 = jnp.maximum(jnp.max(a1t, axis=1, keepdims=True), m1s[...])
    c1 = jnp.exp(m1s[...] - t1)
    p1 = jnp.exp(a1t - t1)
    l1s[...] = l1s[...] * c1 + jnp.sum(p1, axis=1, keepdims=True)
    acc1[...] = acc1[...] * c1 + _dot(p1, h1)
    m1s[...] = t1

    t2 = jnp.maximum(jnp.max(a2t, axis=1, keepdims=True), m2s[...])
    c2 = jnp.exp(m2s[...] - t2)
    p2 = jnp.exp(a2t - t2)
    l2s[...] = l2s[...] * c2 + jnp.sum(p2, axis=1, keepdims=True)
    acc2[...] = acc2[...] * c2 + _dot(p2, h2)
    m2s[...] = t2

    @pl.when(i == NGRID - 1)
    def _fin():
        m1_out[...] = acc1[...] / l1s[...]
        m2_out[...] = acc2[...] / l2s[...]


def _tail_kernel(m1_ref, m2_ref,
                 e1w1_ref, e1b1_ref, e1w2_ref, e1b2_ref,
                 e3w1_ref, e3b1_ref, e3w2_ref, e3b2_ref,
                 e2w1_ref, e2b1_ref, e2w2_ref, e2b2_ref, e2pw_ref, e2pb_ref,
                 gw1_ref, gb1_ref, gw2_ref, gb2_ref,
                 fw_ref, fb_ref, cw_ref, cb_ref,
                 logits_out, prob_out, yhat_out):
    m1 = m1_ref[...]
    m2 = m2_ref[...]
    relu = lambda v: jnp.maximum(v, 0.0)
    e1 = relu(_dot(relu(_dot(m1, e1w1_ref[...]) + e1b1_ref[...]),
                   e1w2_ref[...]) + e1b2_ref[...])
    e3 = relu(_dot(relu(_dot(m2, e3w1_ref[...]) + e3b1_ref[...]),
                   e3w2_ref[...]) + e3b2_ref[...])
    t = relu(_dot(m1, e2w1_ref[0:512, :]) + _dot(m2, e2w1_ref[512:1024, :])
             + e2b1_ref[...])
    e2 = relu(_dot(t, e2w2_ref[...]) + e2b2_ref[...])
    e2 = _dot(e2, e2pw_ref[...]) + e2pb_ref[...]
    g = (_dot(e1, gw1_ref[0:512, :]) + _dot(e2, gw1_ref[512:1024, :])
         + _dot(e3, gw1_ref[1024:1536, :]) + gb1_ref[...])
    g = _dot(relu(g), gw2_ref[...]) + gb2_ref[...]
    g = g - jnp.max(g, axis=1, keepdims=True)
    g = jnp.exp(g)
    g = g / jnp.sum(g, axis=1, keepdims=True)
    fused = (g[:, 0:1] * e1 + g[:, 1:2] * e2 + g[:, 2:3] * e3
             + _dot(m1, fw_ref[0:512, :]) + _dot(m2, fw_ref[512:1024, :])
             + fb_ref[...])
    logits = _dot(fused, cw_ref[...]) + cb_ref[...]
    logits_out[...] = logits
    z = logits - jnp.max(logits, axis=1, keepdims=True)
    z = jnp.exp(z)
    prob_out[...] = z / jnp.sum(z, axis=1, keepdims=True)
    yhat_out[...] = (logits[:, 1:2] > logits[:, 0:1]).astype(jnp.int32)


def _full(shape):
    return pl.BlockSpec(shape, lambda i: (0, 0))


def kernel(x1, x2, params):
    p = params
    row = lambda v: v.reshape(1, -1)
    col = lambda v: v.reshape(-1, 1)

    stream_in = [
        x1, x2,
        p['proj_virchow_w'], row(p['proj_virchow_b']),
        p['fc1_w'], row(p['fc1_b']),
        p['attn1_a_w'], row(p['attn1_a_b']),
        p['attn1_b_w'], row(p['attn1_b_b']),
        p['attn1_c_w'], col(p['attn1_c_b']),
        p['fc2_w'], row(p['fc2_b']),
        p['attn2_a_w'], row(p['attn2_a_b']),
        p['attn2_b_w'], row(p['attn2_b_b']),
        p['attn2_c_w'], col(p['attn2_c_b']),
    ]
    stream_specs = [
        pl.BlockSpec((TILE, 2560), lambda i: (i, 0)),
        pl.BlockSpec((TILE, 1024), lambda i: (i, 0)),
    ] + [_full(a.shape) for a in stream_in[2:]]

    a1_raw, a2_raw, m1, m2 = pl.pallas_call(
        _stream_kernel,
        grid=(NGRID,),
        in_specs=stream_specs,
        out_specs=[
            pl.BlockSpec((2, TILE), lambda i: (0, i)),
            pl.BlockSpec((2, TILE), lambda i: (0, i)),
            _full((2, 512)),
            _full((2, 512)),
        ],
        out_shape=[
            jax.ShapeDtypeStruct((2, N), F32),
            jax.ShapeDtypeStruct((2, N), F32),
            jax.ShapeDtypeStruct((2, 512), F32),
            jax.ShapeDtypeStruct((2, 512), F32),
        ],
        scratch_shapes=[
            pltpu.VMEM((2, 1), F32), pltpu.VMEM((2, 1), F32),
            pltpu.VMEM((2, 512), F32),
            pltpu.VMEM((2, 1), F32), pltpu.VMEM((2, 1), F32),
            pltpu.VMEM((2, 512), F32),
        ],
    )(*stream_in)

    tail_in = [
        m1, m2,
        p['e1_w1'], row(p['e1_b1']), p['e1_w2'], row(p['e1_b2']),
        p['e3_w1'], row(p['e3_b1']), p['e3_w2'], row(p['e3_b2']),
        p['e2_w1'], row(p['e2_b1']), p['e2_w2'], row(p['e2_b2']),
        p['e2_proj_w'], row(p['e2_proj_b']),
        p['gate_w1'], row(p['gate_b1']), p['gate_w2'], row(p['gate_b2']),
        p['fusion_w'], row(p['fusion_b']),
        p['cls_w'], row(p['cls_b']),
    ]
    logits, y_prob, yhat = pl.pallas_call(
        _tail_kernel,
        out_shape=[
            jax.ShapeDtypeStruct((2, 2), F32),
            jax.ShapeDtypeStruct((2, 2), F32),
            jax.ShapeDtypeStruct((2, 1), jnp.int32),
        ],
    )(*tail_in)

    return (logits, y_prob, yhat.reshape(2), a1_raw, a2_raw)


# speedup vs baseline: 1.1176x; 1.0135x over previous
import jax
import jax.numpy as jnp
from jax.experimental import pallas as pl
from jax.experimental.pallas import tpu as pltpu

N = 8192
TILE = 512
NGRID = N // TILE
F32 = jnp.float32

_TAIL_W_SHAPES = [
    (512, 1024), (1024, 512),
    (512, 1024), (1024, 512),
    (1024, 2048), (2048, 1024),
    (1024, 512),
    (1536, 256), (256, 3),
    (1024, 512),
    (512, 2),
]
NTW = len(_TAIL_W_SHAPES)


def _dot(a, b):
    return jax.lax.dot(a, b, preferred_element_type=F32)


def _dot_t(w, x):
    return jax.lax.dot_general(w, x, (((0,), (1,)), ((), ())),
                               preferred_element_type=F32)


def _stream_kernel(*refs):
    (x1_ref, x2_ref,
     pvw_ref, pvb_ref, fc1w_ref, fc1b_ref,
     a1aw_ref, a1ab_ref, a1bw_ref, a1bb_ref, a1cw_ref, a1cb_ref,
     fc2w_ref, fc2b_ref,
     a2aw_ref, a2ab_ref, a2bw_ref, a2bb_ref, a2cw_ref, a2cb_ref) = refs[:20]
    tail_hbm = refs[20:20 + NTW]
    (e1b1_ref, e1b2_ref, e3b1_ref, e3b2_ref, e2b1_ref, e2b2_ref,
     e2pb_ref, gb1_ref, gb2_ref, fb_ref, cb_ref) = refs[20 + NTW:31 + NTW]
    a1_out, a2_out, logits_out, prob_out, yhat_out = refs[31 + NTW:36 + NTW]
    scratch = refs[36 + NTW:]
    m1s, l1s, acc1, m2s, l2s, acc2 = scratch[:6]
    tail_vmem = scratch[6:6 + NTW]
    sems = scratch[6 + NTW:]

    i = pl.program_id(0)

    @pl.when(i == 0)
    def _init():
        m1s[...] = jnp.full((2, 1), -jnp.inf, F32)
        l1s[...] = jnp.zeros((2, 1), F32)
        acc1[...] = jnp.zeros((2, 512), F32)
        m2s[...] = jnp.full((2, 1), -jnp.inf, F32)
        l2s[...] = jnp.zeros((2, 1), F32)
        acc2[...] = jnp.zeros((2, 512), F32)
        for k in range(NTW):
            pltpu.make_async_copy(tail_hbm[k], tail_vmem[k], sems[k]).start()

    h1 = _dot(x1_ref[...], pvw_ref[...]) + pvb_ref[...]
    h1 = jnp.maximum(_dot(h1, fc1w_ref[...]) + fc1b_ref[...], 0.0)
    a = jnp.tanh(_dot(h1, a1aw_ref[...]) + a1ab_ref[...])
    b = jax.nn.sigmoid(_dot(h1, a1bw_ref[...]) + a1bb_ref[...])
    a1t = _dot_t(a1cw_ref[...], a * b) + a1cb_ref[...]
    a1_out[...] = a1t

    h2 = jnp.maximum(_dot(x2_ref[...], fc2w_ref[...]) + fc2b_ref[...], 0.0)
    a = jnp.tanh(_dot(h2, a2aw_ref[...]) + a2ab_ref[...])
    b = jax.nn.sigmoid(_dot(h2, a2bw_ref[...]) + a2bb_ref[...])
    a2t = _dot_t(a2cw_ref[...], a * b) + a2cb_ref[...]
    a2_out[...] = a2t

    t1 = jnp.maximum(jnp.max(a1t, axis=1, keepdims=True), m1s[...])
    c1 = jnp.exp(m1s[...] - t1)
    p1 = jnp.exp(a1t - t1)
    l1s[...] = l1s[...] * c1 + jnp.sum(p1, axis=1, keepdims=True)
    acc1[...] = acc1[...] * c1 + _dot(p1, h1)
    m1s[...] = t1

    t2 = jnp.maximum(jnp.max(a2t, axis=1, keepdims=True), m2s[...])
    c2 = jnp.exp(m2s[...] - t2)
    p2 = jnp.exp(a2t - t2)
    l2s[...] = l2s[...] * c2 + jnp.sum(p2, axis=1, keepdims=True)
    acc2[...] = acc2[...] * c2 + _dot(p2, h2)
    m2s[...] = t2

    @pl.when(i == NGRID - 1)
    def _tail():
        for k in range(NTW):
            pltpu.make_async_copy(tail_hbm[k], tail_vmem[k], sems[k]).wait()
        (e1w1, e1w2, e3w1, e3w2, e2w1, e2w2, e2pw, gw1, gw2, fw, cw) = (
            r[...] for r in tail_vmem)
        m1 = acc1[...] / l1s[...]
        m2 = acc2[...] / l2s[...]
        relu = lambda v: jnp.maximum(v, 0.0)
        e1 = relu(_dot(relu(_dot(m1, e1w1) + e1b1_ref[...]), e1w2)
                  + e1b2_ref[...])
        e3 = relu(_dot(relu(_dot(m2, e3w1) + e3b1_ref[...]), e3w2)
                  + e3b2_ref[...])
        t = relu(_dot(m1, e2w1[0:512, :]) + _dot(m2, e2w1[512:1024, :])
                 + e2b1_ref[...])
        e2 = relu(_dot(t, e2w2) + e2b2_ref[...])
        e2 = _dot(e2, e2pw) + e2pb_ref[...]
        g = (_dot(e1, gw1[0:512, :]) + _dot(e2, gw1[512:1024, :])
             + _dot(e3, gw1[1024:1536, :]) + gb1_ref[...])
        g = _dot(relu(g), gw2) + gb2_ref[...]
        g = g - jnp.max(g, axis=1, keepdims=True)
        g = jnp.exp(g)
        g = g / jnp.sum(g, axis=1, keepdims=True)
        fused = (g[:, 0:1] * e1 + g[:, 1:2] * e2 + g[:, 2:3] * e3
                 + _dot(m1, fw[0:512, :]) + _dot(m2, fw[512:1024, :])
                 + fb_ref[...])
        logits = _dot(fused, cw) + cb_ref[...]
        logits_out[...] = logits
        z = logits - jnp.max(logits, axis=1, keepdims=True)
        z = jnp.exp(z)
        prob_out[...] = z / jnp.sum(z, axis=1, keepdims=True)
        yhat_out[...] = (logits[:, 1:2] > logits[:, 0:1]).astype(jnp.int32)


def _full(shape):
    return pl.BlockSpec(shape, lambda i: tuple(0 for _ in shape))


def kernel(x1, x2, params):
    p = params
    row = lambda v: v.reshape(1, -1)
    col = lambda v: v.reshape(-1, 1)

    ins = [
        x1, x2,
        p['proj_virchow_w'], row(p['proj_virchow_b']),
        p['fc1_w'], row(p['fc1_b']),
        p['attn1_a_w'], row(p['attn1_a_b']),
        p['attn1_b_w'], row(p['attn1_b_b']),
        p['attn1_c_w'], col(p['attn1_c_b']),
        p['fc2_w'], row(p['fc2_b']),
        p['attn2_a_w'], row(p['attn2_a_b']),
        p['attn2_b_w'], row(p['attn2_b_b']),
        p['attn2_c_w'], col(p['attn2_c_b']),
        p['e1_w1'], p['e1_w2'], p['e3_w1'], p['e3_w2'],
        p['e2_w1'], p['e2_w2'], p['e2_proj_w'],
        p['gate_w1'], p['gate_w2'], p['fusion_w'], p['cls_w'],
        row(p['e1_b1']), row(p['e1_b2']), row(p['e3_b1']), row(p['e3_b2']),
        row(p['e2_b1']), row(p['e2_b2']), row(p['e2_proj_b']),
        row(p['gate_b1']), row(p['gate_b2']), row(p['fusion_b']),
        row(p['cls_b']),
    ]
    in_specs = (
        [pl.BlockSpec((TILE, 2560), lambda i: (i, 0)),
         pl.BlockSpec((TILE, 1024), lambda i: (i, 0))]
        + [_full(a.shape) for a in ins[2:20]]
        + [pl.BlockSpec(memory_space=pl.ANY)] * NTW
        + [_full(a.shape) for a in ins[20 + NTW:]]
    )
    out_specs = [
        pl.BlockSpec((2, TILE), lambda i: (0, i)),
        pl.BlockSpec((2, TILE), lambda i: (0, i)),
        _full((2, 2)), _full((2, 2)), _full((2, 1)),
    ]
    out_shape = [
        jax.ShapeDtypeStruct((2, N), F32),
        jax.ShapeDtypeStruct((2, N), F32),
        jax.ShapeDtypeStruct((2, 2), F32),
        jax.ShapeDtypeStruct((2, 2), F32),
        jax.ShapeDtypeStruct((2, 1), jnp.int32),
    ]
    scratch = (
        [pltpu.VMEM((2, 1), F32), pltpu.VMEM((2, 1), F32),
         pltpu.VMEM((2, 512), F32),
         pltpu.VMEM((2, 1), F32), pltpu.VMEM((2, 1), F32),
         pltpu.VMEM((2, 512), F32)]
        + [pltpu.VMEM(s, F32) for s in _TAIL_W_SHAPES]
        + [pltpu.SemaphoreType.DMA] * NTW
    )
    a1_raw, a2_raw, logits, y_prob, yhat = pl.pallas_call(
        _stream_kernel,
        grid=(NGRID,),
        in_specs=in_specs,
        out_specs=out_specs,
        out_shape=out_shape,
        scratch_shapes=scratch,
        compiler_params=pltpu.CompilerParams(
            vmem_limit_bytes=110 * 1024 * 1024),
    )(*ins)

    return (logits, y_prob, yhat.reshape(2), a1_raw, a2_raw)
